# Initial kernel scaffold; baseline (speedup 1.0000x reference)
import jax
import jax.numpy as jnp
from jax.experimental import pallas as pl


def kernel(sequence, token_table, W, b, pos_table, gamma, beta):
    raise NotImplementedError("write your pallas kernel here")


# speedup vs baseline: 4.6403x; 4.6403x over previous
import functools

import jax
import jax.numpy as jnp
from jax import lax
from jax.experimental import pallas as pl
from jax.experimental.pallas import tpu as pltpu
from jax.experimental.pallas import tpu_sc as plsc

_NC = 2
_NS = 16
_NW = _NC * _NS
_CHUNK = 128


@functools.lru_cache(maxsize=None)
def _make_gather(num_idx: int, vocab: int, embed: int):
    assert num_idx % (_NW * _CHUNK) == 0
    n_per_w = num_idx // _NW
    n_ch = n_per_w // _CHUNK

    mesh = plsc.VectorSubcoreMesh(core_axis_name="c", subcore_axis_name="s")

    @functools.partial(
        pl.kernel,
        out_type=jax.ShapeDtypeStruct((num_idx, embed), jnp.float32),
        mesh=mesh,
        scratch_types=[
            pltpu.VMEM((n_ch, _CHUNK), jnp.int32),
            pltpu.VMEM((n_per_w, embed), jnp.float32),
            pltpu.SemaphoreType.DMA,
        ],
    )
    def gather_kernel(idx_hbm, table_hbm, out_hbm, idx_v, rows_v, sem):
        wid = lax.axis_index("s") * _NC + lax.axis_index("c")
        pltpu.sync_copy(idx_hbm.at[pl.ds(wid * n_ch, n_ch)], idx_v)
        copies = [
            pltpu.async_copy(
                table_hbm.at[idx_v.at[j]],
                rows_v.at[pl.ds(j * _CHUNK, _CHUNK)],
                sem,
            )
            for j in range(n_ch)
        ]
        for c in copies:
            c.wait()
        pltpu.sync_copy(rows_v, out_hbm.at[pl.ds(wid * n_per_w, n_per_w)])

    return gather_kernel


@functools.lru_cache(maxsize=None)
def _make_dense(batch: int, seq: int, embed: int, hidden: int, t_blk: int):
    assert seq % t_blk == 0
    grid = (seq // t_blk, batch)

    def body(x_ref, w_ref, b_ref, p_ref, g_ref, be_ref, o_ref):
        x = x_ref[0]
        y = jnp.dot(x, w_ref[...], preferred_element_type=jnp.float32)
        y = y + b_ref[...] + p_ref[...]
        mean = jnp.mean(y, axis=-1, keepdims=True)
        yc = y - mean
        var = jnp.mean(yc * yc, axis=-1, keepdims=True)
        o_ref[0] = (g_ref[...] * lax.rsqrt(var + 1e-6)) * yc + be_ref[...]

    return pl.pallas_call(
        body,
        grid=grid,
        in_specs=[
            pl.BlockSpec((1, t_blk, embed), lambda j, i: (i, j, 0)),
            pl.BlockSpec((embed, hidden), lambda j, i: (0, 0)),
            pl.BlockSpec((1, hidden), lambda j, i: (0, 0)),
            pl.BlockSpec((t_blk, hidden), lambda j, i: (j, 0)),
            pl.BlockSpec((1, hidden), lambda j, i: (0, 0)),
            pl.BlockSpec((1, hidden), lambda j, i: (0, 0)),
        ],
        out_specs=pl.BlockSpec((1, t_blk, hidden), lambda j, i: (i, j, 0)),
        out_shape=jax.ShapeDtypeStruct((batch, seq, hidden), jnp.float32),
    )


def kernel(sequence, token_table, W, b, pos_table, gamma, beta):
    batch, seq = sequence.shape
    vocab, embed = token_table.shape
    hidden = W.shape[1]
    n = batch * seq

    idx = sequence.astype(jnp.int32).reshape(n // _CHUNK, _CHUNK)
    gathered = _make_gather(n, vocab, embed)(idx, token_table)
    x3 = gathered.reshape(batch, seq, embed)

    dense = _make_dense(batch, seq, embed, hidden, 512)
    return dense(
        x3,
        W,
        b.reshape(1, hidden),
        pos_table[:seq],
        gamma.reshape(1, hidden),
        beta.reshape(1, hidden),
    )


# speedup vs baseline: 5.1164x; 1.1026x over previous
import functools

import jax
import jax.numpy as jnp
from jax import lax
from jax.experimental import pallas as pl
from jax.experimental.pallas import tpu as pltpu
from jax.experimental.pallas import tpu_sc as plsc

_NC = 2
_NS = 16
_NW = _NC * _NS
_CHUNK = 128


@functools.lru_cache(maxsize=None)
def _make_gather(num_idx: int, vocab: int, embed: int):
    assert num_idx % (_NW * _CHUNK) == 0
    n_per_w = num_idx // _NW
    n_ch = n_per_w // _CHUNK

    mesh = plsc.VectorSubcoreMesh(core_axis_name="c", subcore_axis_name="s")

    @functools.partial(
        pl.kernel,
        out_type=jax.ShapeDtypeStruct((num_idx, embed), jnp.float32),
        mesh=mesh,
        scratch_types=[
            pltpu.VMEM((n_ch, _CHUNK), jnp.int32),
            pltpu.VMEM((n_per_w, embed), jnp.float32),
            pltpu.SemaphoreType.DMA,
        ],
    )
    def gather_kernel(idx_hbm, table_hbm, out_hbm, idx_v, rows_v, sem):
        wid = lax.axis_index("s") * _NC + lax.axis_index("c")
        pltpu.sync_copy(idx_hbm.at[pl.ds(wid * n_ch, n_ch)], idx_v)
        copies = [
            pltpu.async_copy(
                table_hbm.at[idx_v.at[j]],
                rows_v.at[pl.ds(j * _CHUNK, _CHUNK)],
                sem,
            )
            for j in range(n_ch)
        ]
        for c in copies:
            c.wait()
        pltpu.sync_copy(rows_v, out_hbm.at[pl.ds(wid * n_per_w, n_per_w)])

    return gather_kernel


@functools.lru_cache(maxsize=None)
def _make_dense(batch: int, seq: int, embed: int, hidden: int, t_blk: int):
    assert seq % t_blk == 0
    grid = (seq // t_blk, batch)

    def body(x_ref, w_ref, b_ref, p_ref, g_ref, be_ref, o_ref):
        x = x_ref[0]
        y = jnp.dot(x, w_ref[...], preferred_element_type=jnp.float32)
        y = y + b_ref[...] + p_ref[...]
        mean = jnp.mean(y, axis=-1, keepdims=True)
        yc = y - mean
        var = jnp.mean(yc * yc, axis=-1, keepdims=True)
        o_ref[0] = (g_ref[...] * lax.rsqrt(var + 1e-6)) * yc + be_ref[...]

    return pl.pallas_call(
        body,
        grid=grid,
        in_specs=[
            pl.BlockSpec((1, t_blk, embed), lambda j, i: (i, j, 0)),
            pl.BlockSpec((embed, hidden), lambda j, i: (0, 0)),
            pl.BlockSpec((1, hidden), lambda j, i: (0, 0)),
            pl.BlockSpec((t_blk, hidden), lambda j, i: (j, 0)),
            pl.BlockSpec((1, hidden), lambda j, i: (0, 0)),
            pl.BlockSpec((1, hidden), lambda j, i: (0, 0)),
        ],
        out_specs=pl.BlockSpec((1, t_blk, hidden), lambda j, i: (i, j, 0)),
        out_shape=jax.ShapeDtypeStruct((batch, seq, hidden), jnp.float32),
    )


def kernel(sequence, token_table, W, b, pos_table, gamma, beta):
    batch, seq = sequence.shape
    vocab, embed = token_table.shape
    hidden = W.shape[1]
    n = batch * seq

    idx = sequence.astype(jnp.int32).reshape(n // _CHUNK, _CHUNK)
    gathered = _make_gather(n, vocab, embed)(idx, token_table)
    x3 = gathered.reshape(batch, seq, embed)

    dense = _make_dense(batch, seq, embed, hidden, 1024)
    return dense(
        x3,
        W,
        b.reshape(1, hidden),
        pos_table[:seq],
        gamma.reshape(1, hidden),
        beta.reshape(1, hidden),
    )


# speedup vs baseline: 5.4242x; 1.0601x over previous
import functools

import jax
import jax.numpy as jnp
from jax import lax
from jax.experimental import pallas as pl
from jax.experimental.pallas import tpu as pltpu
from jax.experimental.pallas import tpu_sc as plsc

_NC = 2
_NS = 16
_NW = _NC * _NS
_CHUNK = 128


@functools.lru_cache(maxsize=None)
def _make_gather(num_idx: int, vocab: int, embed: int):
    assert num_idx % (_NW * _CHUNK) == 0
    n_per_w = num_idx // _NW
    n_ch = n_per_w // _CHUNK

    mesh = plsc.VectorSubcoreMesh(core_axis_name="c", subcore_axis_name="s")

    @functools.partial(
        pl.kernel,
        out_type=jax.ShapeDtypeStruct((num_idx, embed), jnp.float32),
        mesh=mesh,
        scratch_types=[
            pltpu.VMEM((n_ch, _CHUNK), jnp.int32),
            pltpu.VMEM((n_per_w, embed), jnp.float32),
            pltpu.SemaphoreType.DMA,
        ],
    )
    def gather_kernel(idx_hbm, table_hbm, out_hbm, idx_v, rows_v, sem):
        wid = lax.axis_index("s") * _NC + lax.axis_index("c")
        pltpu.sync_copy(idx_hbm.at[pl.ds(wid * n_ch, n_ch)], idx_v)
        copies = [
            pltpu.async_copy(
                table_hbm.at[idx_v.at[j]],
                rows_v.at[pl.ds(j * _CHUNK, _CHUNK)],
                sem,
            )
            for j in range(n_ch)
        ]
        for c in copies:
            c.wait()
        pltpu.sync_copy(rows_v, out_hbm.at[pl.ds(wid * n_per_w, n_per_w)])

    return gather_kernel


@functools.lru_cache(maxsize=None)
def _make_dense(batch: int, seq: int, embed: int, hidden: int, t_blk: int):
    assert seq % t_blk == 0
    grid = (seq // t_blk, batch)

    def body(x_ref, w_ref, b_ref, p_ref, g_ref, be_ref, o_ref):
        x = x_ref[0]
        y = jnp.dot(x, w_ref[...], preferred_element_type=jnp.float32)
        y = y + b_ref[...] + p_ref[...]
        mean = jnp.mean(y, axis=-1, keepdims=True)
        yc = y - mean
        var = jnp.mean(yc * yc, axis=-1, keepdims=True)
        o_ref[0] = (g_ref[...] * lax.rsqrt(var + 1e-6)) * yc + be_ref[...]

    return pl.pallas_call(
        body,
        grid=grid,
        in_specs=[
            pl.BlockSpec((1, t_blk, embed), lambda j, i: (i, j, 0)),
            pl.BlockSpec((embed, hidden), lambda j, i: (0, 0)),
            pl.BlockSpec((1, hidden), lambda j, i: (0, 0)),
            pl.BlockSpec((t_blk, hidden), lambda j, i: (j, 0)),
            pl.BlockSpec((1, hidden), lambda j, i: (0, 0)),
            pl.BlockSpec((1, hidden), lambda j, i: (0, 0)),
        ],
        out_specs=pl.BlockSpec((1, t_blk, hidden), lambda j, i: (i, j, 0)),
        out_shape=jax.ShapeDtypeStruct((batch, seq, hidden), jnp.float32),
    )


def kernel(sequence, token_table, W, b, pos_table, gamma, beta):
    batch, seq = sequence.shape
    vocab, embed = token_table.shape
    hidden = W.shape[1]
    n = batch * seq

    idx = sequence.astype(jnp.int32).reshape(n // _CHUNK, _CHUNK)
    gathered = _make_gather(n, vocab, embed)(idx, token_table)
    x3 = gathered.reshape(batch, seq, embed)

    dense = _make_dense(batch, seq, embed, hidden, 2048)
    return dense(
        x3,
        W,
        b.reshape(1, hidden),
        pos_table[:seq],
        gamma.reshape(1, hidden),
        beta.reshape(1, hidden),
    )
